# baseline (speedup 1.0000x reference)
import functools

import jax
import jax.numpy as jnp
from jax import lax
from jax.experimental import pallas as pl
from jax.experimental.pallas import tpu as pltpu
from jax.experimental.pallas import tpu_sc as plsc

TSTEPS = 2000
NB = 256
W = 224 * 224
LANES = 16
BR = 8

_mesh = plsc.VectorSubcoreMesh(core_axis_name="c", subcore_axis_name="s")


@functools.partial(
    pl.kernel,
    out_type=[
        jax.ShapeDtypeStruct((NB,), jnp.float32),
        jax.ShapeDtypeStruct((NB,), jnp.float32),
    ],
    mesh=_mesh,
    scratch_types=[
        pltpu.VMEM((LANES,), jnp.int32),
        pltpu.VMEM((LANES,), jnp.float32),
        pltpu.VMEM((LANES,), jnp.float32),
        pltpu.SemaphoreType.DMA,
    ],
)
def _gather_coefs(t_hbm, g_hbm, s_hbm, outg_hbm, outs_hbm,
                  idx_v, gbuf_v, sbuf_v, sem):
    wid = lax.axis_index("s") * 2 + lax.axis_index("c")

    @pl.when(wid < NB // LANES)
    def _():
        base = wid * LANES
        pltpu.sync_copy(t_hbm.at[pl.ds(base, LANES)], idx_v)
        pltpu.async_copy(g_hbm.at[idx_v], gbuf_v, sem).wait()
        pltpu.async_copy(s_hbm.at[idx_v], sbuf_v, sem).wait()
        pltpu.sync_copy(gbuf_v, outg_hbm.at[pl.ds(base, LANES)])
        pltpu.sync_copy(sbuf_v, outs_hbm.at[pl.ds(base, LANES)])


def _scale_add_body(g_ref, s_ref, y_ref, n_ref, oy_ref, on_ref):
    nv = n_ref[...]
    oy_ref[...] = g_ref[...] * y_ref[...] + s_ref[...] * nv
    on_ref[...] = nv


H = 224
IMG4 = (NB, 1, H, H)

_scale_add_call = pl.pallas_call(
    _scale_add_body,
    grid=(NB // BR,),
    in_specs=[
        pl.BlockSpec((BR, 1, 1, 1), lambda i: (i, 0, 0, 0)),
        pl.BlockSpec((BR, 1, 1, 1), lambda i: (i, 0, 0, 0)),
        pl.BlockSpec((BR, 1, H, H), lambda i: (i, 0, 0, 0)),
        pl.BlockSpec((BR, 1, H, H), lambda i: (i, 0, 0, 0)),
    ],
    out_specs=[
        pl.BlockSpec((BR, 1, H, H), lambda i: (i, 0, 0, 0)),
        pl.BlockSpec((BR, 1, H, H), lambda i: (i, 0, 0, 0)),
    ],
    out_shape=[
        jax.ShapeDtypeStruct(IMG4, jnp.float32),
        jax.ShapeDtypeStruct(IMG4, jnp.float32),
    ],
)


def kernel(y, noise, t, gammas, sqrt_one_minus_gammas, sqrt_gammas):
    t32 = t.astype(jnp.int32)
    g_t, s_t = _gather_coefs(t32, gammas, sqrt_one_minus_gammas)
    oy, on = _scale_add_call(
        g_t.reshape(NB, 1, 1, 1), s_t.reshape(NB, 1, 1, 1), y, noise)
    return oy, on


# speedup vs baseline: 1.1398x; 1.1398x over previous
import functools

import jax
import jax.numpy as jnp
from jax import lax
from jax.experimental import pallas as pl
from jax.experimental.pallas import tpu as pltpu
from jax.experimental.pallas import tpu_sc as plsc

TSTEPS = 2000
NB = 256
W = 224 * 224
LANES = 16
BR = 8

_mesh = plsc.VectorSubcoreMesh(core_axis_name="c", subcore_axis_name="s")


@functools.partial(
    pl.kernel,
    out_type=[
        jax.ShapeDtypeStruct((NB,), jnp.float32),
        jax.ShapeDtypeStruct((NB,), jnp.float32),
    ],
    mesh=_mesh,
    scratch_types=[
        pltpu.VMEM((LANES,), jnp.int32),
        pltpu.VMEM((LANES,), jnp.float32),
        pltpu.VMEM((LANES,), jnp.float32),
        pltpu.SemaphoreType.DMA,
    ],
)
def _gather_coefs(t_hbm, g_hbm, s_hbm, outg_hbm, outs_hbm,
                  idx_v, gbuf_v, sbuf_v, sem):
    wid = lax.axis_index("s") * 2 + lax.axis_index("c")

    @pl.when(wid < NB // LANES)
    def _():
        base = wid * LANES
        pltpu.sync_copy(t_hbm.at[pl.ds(base, LANES)], idx_v)
        pltpu.async_copy(g_hbm.at[idx_v], gbuf_v, sem).wait()
        pltpu.async_copy(s_hbm.at[idx_v], sbuf_v, sem).wait()
        pltpu.sync_copy(gbuf_v, outg_hbm.at[pl.ds(base, LANES)])
        pltpu.sync_copy(sbuf_v, outs_hbm.at[pl.ds(base, LANES)])


def _scale_add_body(g_ref, s_ref, y_ref, n_ref, oy_ref):
    oy_ref[...] = g_ref[...] * y_ref[...] + s_ref[...] * n_ref[...]


H = 224
IMG4 = (NB, 1, H, H)

_scale_add_call = pl.pallas_call(
    _scale_add_body,
    grid=(NB // BR,),
    in_specs=[
        pl.BlockSpec((BR, 1, 1, 1), lambda i: (i, 0, 0, 0)),
        pl.BlockSpec((BR, 1, 1, 1), lambda i: (i, 0, 0, 0)),
        pl.BlockSpec((BR, 1, H, H), lambda i: (i, 0, 0, 0)),
        pl.BlockSpec((BR, 1, H, H), lambda i: (i, 0, 0, 0)),
    ],
    out_specs=[
        pl.BlockSpec((BR, 1, H, H), lambda i: (i, 0, 0, 0)),
    ],
    out_shape=[
        jax.ShapeDtypeStruct(IMG4, jnp.float32),
    ],
)


def kernel(y, noise, t, gammas, sqrt_one_minus_gammas, sqrt_gammas):
    t32 = t.astype(jnp.int32)
    g_t, s_t = _gather_coefs(t32, gammas, sqrt_one_minus_gammas)
    (oy,) = _scale_add_call(
        g_t.reshape(NB, 1, 1, 1), s_t.reshape(NB, 1, 1, 1), y, noise)
    return oy, noise


# speedup vs baseline: 1.1543x; 1.0128x over previous
import functools

import jax
import jax.numpy as jnp
from jax import lax
from jax.experimental import pallas as pl
from jax.experimental.pallas import tpu as pltpu
from jax.experimental.pallas import tpu_sc as plsc

TSTEPS = 2000
NB = 256
W = 224 * 224
LANES = 16
BR = 8

_mesh = plsc.VectorSubcoreMesh(core_axis_name="c", subcore_axis_name="s")


@functools.partial(
    pl.kernel,
    out_type=[
        jax.ShapeDtypeStruct((NB,), jnp.float32),
        jax.ShapeDtypeStruct((NB,), jnp.float32),
    ],
    mesh=_mesh,
    scratch_types=[
        pltpu.VMEM((LANES,), jnp.int32),
        pltpu.VMEM((LANES,), jnp.float32),
        pltpu.VMEM((LANES,), jnp.float32),
        pltpu.SemaphoreType.DMA,
    ],
)
def _gather_coefs(t_hbm, g_hbm, s_hbm, outg_hbm, outs_hbm,
                  idx_v, gbuf_v, sbuf_v, sem):
    wid = lax.axis_index("s") * 2 + lax.axis_index("c")

    @pl.when(wid < NB // LANES)
    def _():
        base = wid * LANES
        pltpu.sync_copy(t_hbm.at[pl.ds(base, LANES)], idx_v)
        pltpu.async_copy(g_hbm.at[idx_v], gbuf_v, sem).wait()
        pltpu.async_copy(s_hbm.at[idx_v], sbuf_v, sem).wait()
        pltpu.sync_copy(gbuf_v, outg_hbm.at[pl.ds(base, LANES)])
        pltpu.sync_copy(sbuf_v, outs_hbm.at[pl.ds(base, LANES)])


def _scale_add_body(g_sm, s_sm, y_ref, n_ref, oy_ref):
    i = pl.program_id(0)
    for r in range(BR):
        g = g_sm[i * BR + r]
        s = s_sm[i * BR + r]
        oy_ref[r, 0] = g * y_ref[r, 0] + s * n_ref[r, 0]


H = 224
IMG4 = (NB, 1, H, H)

_scale_add_call = pl.pallas_call(
    _scale_add_body,
    grid_spec=pltpu.PrefetchScalarGridSpec(
        num_scalar_prefetch=2,
        grid=(NB // BR,),
        in_specs=[
            pl.BlockSpec((BR, 1, H, H), lambda i, g, s: (i, 0, 0, 0)),
            pl.BlockSpec((BR, 1, H, H), lambda i, g, s: (i, 0, 0, 0)),
        ],
        out_specs=[
            pl.BlockSpec((BR, 1, H, H), lambda i, g, s: (i, 0, 0, 0)),
        ],
    ),
    out_shape=[
        jax.ShapeDtypeStruct(IMG4, jnp.float32),
    ],
)


def kernel(y, noise, t, gammas, sqrt_one_minus_gammas, sqrt_gammas):
    t32 = t.astype(jnp.int32)
    g_t, s_t = _gather_coefs(t32, gammas, sqrt_one_minus_gammas)
    (oy,) = _scale_add_call(g_t, s_t, y, noise)
    return oy, noise


# speedup vs baseline: 1.1633x; 1.0077x over previous
import functools

import jax
import jax.numpy as jnp
from jax import lax
from jax.experimental import pallas as pl
from jax.experimental.pallas import tpu as pltpu
from jax.experimental.pallas import tpu_sc as plsc

TSTEPS = 2000
NB = 256
W = 224 * 224
LANES = 16
BR = 32

_mesh = plsc.VectorSubcoreMesh(core_axis_name="c", subcore_axis_name="s")


@functools.partial(
    pl.kernel,
    out_type=[
        jax.ShapeDtypeStruct((NB,), jnp.float32),
        jax.ShapeDtypeStruct((NB,), jnp.float32),
    ],
    mesh=_mesh,
    scratch_types=[
        pltpu.VMEM((LANES,), jnp.int32),
        pltpu.VMEM((LANES,), jnp.float32),
        pltpu.VMEM((LANES,), jnp.float32),
        pltpu.SemaphoreType.DMA,
    ],
)
def _gather_coefs(t_hbm, g_hbm, s_hbm, outg_hbm, outs_hbm,
                  idx_v, gbuf_v, sbuf_v, sem):
    wid = lax.axis_index("s") * 2 + lax.axis_index("c")

    @pl.when(wid < NB // LANES)
    def _():
        base = wid * LANES
        pltpu.sync_copy(t_hbm.at[pl.ds(base, LANES)], idx_v)
        pltpu.async_copy(g_hbm.at[idx_v], gbuf_v, sem).wait()
        pltpu.async_copy(s_hbm.at[idx_v], sbuf_v, sem).wait()
        pltpu.sync_copy(gbuf_v, outg_hbm.at[pl.ds(base, LANES)])
        pltpu.sync_copy(sbuf_v, outs_hbm.at[pl.ds(base, LANES)])


def _scale_add_body(g_sm, s_sm, y_ref, n_ref, oy_ref):
    i = pl.program_id(0)
    for r in range(BR):
        g = g_sm[i * BR + r]
        s = s_sm[i * BR + r]
        oy_ref[r, 0] = g * y_ref[r, 0] + s * n_ref[r, 0]


H = 224
IMG4 = (NB, 1, H, H)

_scale_add_call = pl.pallas_call(
    _scale_add_body,
    grid_spec=pltpu.PrefetchScalarGridSpec(
        num_scalar_prefetch=2,
        grid=(NB // BR,),
        in_specs=[
            pl.BlockSpec((BR, 1, H, H), lambda i, g, s: (i, 0, 0, 0)),
            pl.BlockSpec((BR, 1, H, H), lambda i, g, s: (i, 0, 0, 0)),
        ],
        out_specs=[
            pl.BlockSpec((BR, 1, H, H), lambda i, g, s: (i, 0, 0, 0)),
        ],
    ),
    out_shape=[
        jax.ShapeDtypeStruct(IMG4, jnp.float32),
    ],
)


def kernel(y, noise, t, gammas, sqrt_one_minus_gammas, sqrt_gammas):
    t32 = t.astype(jnp.int32)
    g_t, s_t = _gather_coefs(t32, gammas, sqrt_one_minus_gammas)
    (oy,) = _scale_add_call(g_t, s_t, y, noise)
    return oy, noise


# speedup vs baseline: 1.2267x; 1.0546x over previous
import jax
import jax.numpy as jnp
from jax.experimental import pallas as pl
from jax.experimental.pallas import tpu as pltpu

TSTEPS = 2000
NB = 256
H = 224
IMG4 = (NB, 1, H, H)
BR = 8


def _body(t_sm, gam_sm, s1_sm, y_ref, n_ref, oy_ref):
    i = pl.program_id(0)
    for r in range(BR):
        idx = t_sm[i * BR + r]
        g = gam_sm[idx]
        s = s1_sm[idx]
        oy_ref[r, 0] = g * y_ref[r, 0] + s * n_ref[r, 0]


_scale_add_call = pl.pallas_call(
    _body,
    grid_spec=pltpu.PrefetchScalarGridSpec(
        num_scalar_prefetch=3,
        grid=(NB // BR,),
        in_specs=[
            pl.BlockSpec((BR, 1, H, H), lambda i, t, g, s: (i, 0, 0, 0)),
            pl.BlockSpec((BR, 1, H, H), lambda i, t, g, s: (i, 0, 0, 0)),
        ],
        out_specs=[
            pl.BlockSpec((BR, 1, H, H), lambda i, t, g, s: (i, 0, 0, 0)),
        ],
    ),
    out_shape=[
        jax.ShapeDtypeStruct(IMG4, jnp.float32),
    ],
)


def kernel(y, noise, t, gammas, sqrt_one_minus_gammas, sqrt_gammas):
    t32 = t.astype(jnp.int32)
    (oy,) = _scale_add_call(t32, gammas, sqrt_one_minus_gammas, y, noise)
    return oy, noise
